# baseline (speedup 1.0000x reference)
import functools

import jax
import jax.numpy as jnp
from jax import lax
from jax.experimental import pallas as pl
from jax.experimental.pallas import tpu as pltpu
from jax.experimental.pallas import tpu_sc as plsc

NC = 2
NS = 16
LANES = 16
IDX_CHUNK = 128


def _mf_body(n_factors, b_per_w, user_hbm, item_hbm, uf_hbm, if_hbm, out_hbm,
             uidx_v, iidx_v, urows_v, vrows_v, out_v, sem):
    wid = lax.axis_index("s") * NC + lax.axis_index("c")
    n_chunks = b_per_w // IDX_CHUNK

    for j in range(n_chunks):
        src = pl.ds(wid * b_per_w + j * IDX_CHUNK, IDX_CHUNK)
        pltpu.sync_copy(user_hbm.at[src], uidx_v.at[j])
        pltpu.sync_copy(item_hbm.at[src], iidx_v.at[j])

    copies = []
    for j in range(n_chunks):
        dst = pl.ds(j * IDX_CHUNK, IDX_CHUNK)
        copies.append(pltpu.async_copy(uf_hbm.at[uidx_v.at[j]],
                                       urows_v.at[dst], sem))
        copies.append(pltpu.async_copy(if_hbm.at[iidx_v.at[j]],
                                       vrows_v.at[dst], sem))
    for c in copies:
        c.wait()

    def group_body(g, carry):
        rows = g * LANES + lax.iota(jnp.int32, LANES)
        acc = jnp.zeros((LANES,), jnp.float32)
        for f in range(n_factors):
            cols = jnp.full((LANES,), f, jnp.int32)
            uu = plsc.load_gather(urows_v, [rows, cols])
            vv = plsc.load_gather(vrows_v, [rows, cols])
            acc = acc + uu * vv
        out_v[pl.ds(g * LANES, LANES)] = acc
        return carry

    lax.fori_loop(0, b_per_w // LANES, group_body, 0)
    pltpu.sync_copy(out_v, out_hbm.at[pl.ds(wid * b_per_w, b_per_w)])


def kernel(user, item, user_factors, item_factors):
    batch = user.shape[0]
    n_factors = user_factors.shape[1]
    nw = NC * NS
    b_per_w = batch // nw
    n_chunks = b_per_w // IDX_CHUNK

    user_i = user.astype(jnp.int32)
    item_i = item.astype(jnp.int32)

    mesh = plsc.VectorSubcoreMesh(core_axis_name="c", subcore_axis_name="s")
    run = pl.kernel(
        functools.partial(_mf_body, n_factors, b_per_w),
        out_type=jax.ShapeDtypeStruct((batch,), jnp.float32),
        mesh=mesh,
        scratch_types=[
            pltpu.VMEM((n_chunks, IDX_CHUNK), jnp.int32),
            pltpu.VMEM((n_chunks, IDX_CHUNK), jnp.int32),
            pltpu.VMEM((b_per_w, n_factors), jnp.float32),
            pltpu.VMEM((b_per_w, n_factors), jnp.float32),
            pltpu.VMEM((b_per_w,), jnp.float32),
            pltpu.SemaphoreType.DMA,
        ],
        compiler_params=pltpu.CompilerParams(
            needs_layout_passes=False, use_tc_tiling_on_sc=False),
    )
    return run(user_i, item_i, user_factors, item_factors)


# speedup vs baseline: 1.2032x; 1.2032x over previous
import functools

import jax
import jax.numpy as jnp
from jax import lax
from jax.experimental import pallas as pl
from jax.experimental.pallas import tpu as pltpu
from jax.experimental.pallas import tpu_sc as plsc

NC = 2
NS = 16
LANES = 16
IDX_CHUNK = 128


def _mf_body(n_factors, b_per_w, user_hbm, item_hbm, uf_hbm, if_hbm, out_hbm,
             uidx_v, iidx_v, urows_v, vrows_v, out_v, sem):
    wid = lax.axis_index("s") * NC + lax.axis_index("c")
    n_chunks = b_per_w // IDX_CHUNK

    for j in range(n_chunks):
        src = pl.ds(wid * b_per_w + j * IDX_CHUNK, IDX_CHUNK)
        pltpu.sync_copy(user_hbm.at[src], uidx_v.at[j])
        pltpu.sync_copy(item_hbm.at[src], iidx_v.at[j])

    copies = []
    for j in range(n_chunks):
        dst = pl.ds(j * IDX_CHUNK, IDX_CHUNK)
        copies.append(pltpu.async_copy(uf_hbm.at[uidx_v.at[j]],
                                       urows_v.at[dst], sem))
        copies.append(pltpu.async_copy(if_hbm.at[iidx_v.at[j]],
                                       vrows_v.at[dst], sem))
    for c in copies:
        c.wait()

    lane = lax.iota(jnp.int32, LANES)

    def group_body(g, carry):
        rows = g * LANES + lane
        acc = jnp.zeros((LANES,), jnp.float32)
        for f in range(n_factors):
            cols = jnp.bitwise_and(f + lane, n_factors - 1)
            uu = plsc.load_gather(urows_v, [rows, cols])
            vv = plsc.load_gather(vrows_v, [rows, cols])
            acc = acc + uu * vv
        out_v[pl.ds(g * LANES, LANES)] = acc
        return carry

    lax.fori_loop(0, b_per_w // LANES, group_body, 0)
    pltpu.sync_copy(out_v, out_hbm.at[pl.ds(wid * b_per_w, b_per_w)])


def kernel(user, item, user_factors, item_factors):
    batch = user.shape[0]
    n_factors = user_factors.shape[1]
    nw = NC * NS
    b_per_w = batch // nw
    n_chunks = b_per_w // IDX_CHUNK

    user_i = user.astype(jnp.int32)
    item_i = item.astype(jnp.int32)

    mesh = plsc.VectorSubcoreMesh(core_axis_name="c", subcore_axis_name="s")
    run = pl.kernel(
        functools.partial(_mf_body, n_factors, b_per_w),
        out_type=jax.ShapeDtypeStruct((batch,), jnp.float32),
        mesh=mesh,
        scratch_types=[
            pltpu.VMEM((n_chunks, IDX_CHUNK), jnp.int32),
            pltpu.VMEM((n_chunks, IDX_CHUNK), jnp.int32),
            pltpu.VMEM((b_per_w, n_factors), jnp.float32),
            pltpu.VMEM((b_per_w, n_factors), jnp.float32),
            pltpu.VMEM((b_per_w,), jnp.float32),
            pltpu.SemaphoreType.DMA,
        ],
        compiler_params=pltpu.CompilerParams(
            needs_layout_passes=False, use_tc_tiling_on_sc=False),
    )
    return run(user_i, item_i, user_factors, item_factors)


# speedup vs baseline: 1.9059x; 1.5840x over previous
import functools

import jax
import jax.numpy as jnp
from jax import lax
from jax.experimental import pallas as pl
from jax.experimental.pallas import tpu as pltpu
from jax.experimental.pallas import tpu_sc as plsc

NC = 2
NS = 16
LANES = 16
IDX_HALF = 8192


def _pp_body(n_factors, n_rows, batch, user_hbm, item_hbm, ut_hbm, vt_hbm,
             pp_hbm, row_v, val_v, idx_v):
    c = lax.axis_index("c")
    s = lax.axis_index("s")
    n_halves = batch // IDX_HALF
    f_per_tile = n_factors // (NC * NS)

    def gather_phase(table_hbm, idx_hbm, f, combine):
        pltpu.sync_copy(table_hbm.at[f], row_v)
        for h in range(n_halves):
            pltpu.sync_copy(idx_hbm.at[pl.ds(h * IDX_HALF, IDX_HALF)], idx_v)

            def chunk(g, carry):
                cols = idx_v[pl.ds(g * LANES, LANES)]
                got = plsc.load_gather(row_v, [cols])
                slot = pl.ds(h * IDX_HALF + g * LANES, LANES)
                if combine:
                    val_v[slot] = val_v[slot] * got
                else:
                    val_v[slot] = got
                return carry

            lax.fori_loop(0, IDX_HALF // LANES, chunk, 0)

    for k in range(f_per_tile):
        f = c * (n_factors // NC) + s * f_per_tile + k
        gather_phase(ut_hbm, user_hbm, f, combine=False)
        gather_phase(vt_hbm, item_hbm, f, combine=True)
        pltpu.sync_copy(val_v, pp_hbm.at[f])


def _reduce_body(n_factors, b_per_w, pp_hbm, out_hbm, blk_v, out_v):
    wid = lax.axis_index("s") * NC + lax.axis_index("c")
    base = wid * b_per_w
    pltpu.sync_copy(pp_hbm.at[:, pl.ds(base, b_per_w)], blk_v)

    def group(g, carry):
        acc = jnp.zeros((LANES,), jnp.float32)
        for f in range(n_factors):
            acc = acc + blk_v[f, pl.ds(g * LANES, LANES)]
        out_v[pl.ds(g * LANES, LANES)] = acc
        return carry

    lax.fori_loop(0, b_per_w // LANES, group, 0)
    pltpu.sync_copy(out_v, out_hbm.at[pl.ds(base, b_per_w)])


def kernel(user, item, user_factors, item_factors):
    batch = user.shape[0]
    n_rows, n_factors = user_factors.shape
    nw = NC * NS
    b_per_w = batch // nw

    user_i = user.astype(jnp.int32)
    item_i = item.astype(jnp.int32)
    ut = user_factors.T
    vt = item_factors.T

    mesh = plsc.VectorSubcoreMesh(core_axis_name="c", subcore_axis_name="s")
    params = pltpu.CompilerParams(
        needs_layout_passes=False, use_tc_tiling_on_sc=True)

    pp = pl.kernel(
        functools.partial(_pp_body, n_factors, n_rows, batch),
        out_type=jax.ShapeDtypeStruct((n_factors, batch), jnp.float32),
        mesh=mesh,
        scratch_types=[
            pltpu.VMEM((n_rows,), jnp.float32),
            pltpu.VMEM((batch,), jnp.float32),
            pltpu.VMEM((IDX_HALF,), jnp.int32),
        ],
        compiler_params=params,
    )(user_i, item_i, ut, vt)

    out = pl.kernel(
        functools.partial(_reduce_body, n_factors, b_per_w),
        out_type=jax.ShapeDtypeStruct((batch,), jnp.float32),
        mesh=mesh,
        scratch_types=[
            pltpu.VMEM((n_factors, b_per_w), jnp.float32),
            pltpu.VMEM((b_per_w,), jnp.float32),
        ],
        compiler_params=params,
    )(pp)
    return out
